# baseline (speedup 1.0000x reference)
import functools

import jax
import jax.numpy as jnp
from jax import lax
from jax.experimental import pallas as pl
from jax.experimental.pallas import tpu as pltpu
from jax.experimental.pallas import tpu_sc as plsc

N = 10000
E = 320000
D = 128
H2 = 32
C = 16
MAX_LOGSTD = 10.0
EPS = 1e-15

N8 = N + 8
NW = 32
CHUNK = 128
KC = 80
PT = KC * CHUNK
EP = NW * PT
EC = E // CHUNK
GK = 10
GC = GK * CHUNK

_mesh = plsc.VectorSubcoreMesh(core_axis_name="c", subcore_axis_name="s")
_sc_params = pltpu.CompilerParams(use_tc_tiling_on_sc=False,
                                  needs_layout_passes=False)


def _wid():
    return lax.axis_index("s") * 2 + lax.axis_index("c")


def _ngroups(w):
    return jnp.minimum(PT, E - w * PT) // GC


@functools.partial(
    pl.kernel,
    out_type=jax.ShapeDtypeStruct((2, N8), jnp.float32),
    mesh=_mesh,
    compiler_params=_sc_params,
    scratch_types=[
        pltpu.VMEM((GK, CHUNK), jnp.int32),
        pltpu.VMEM((CHUNK,), jnp.float32),
        pltpu.VMEM_SHARED((N8,), jnp.float32),
    ],
)
def _sc_degree(ei_hbm, zeros1_hbm, ones_hbm, out_hbm, idx_v, ones_v, acc_sh):
    c = lax.axis_index("c")
    s = lax.axis_index("s")
    w = _wid()
    pltpu.sync_copy(ones_hbm, ones_v)

    @pl.when(s == 0)
    def _():
        pltpu.sync_copy(zeros1_hbm, acc_sh)

    plsc.subcore_barrier()

    def group(g, carry):
        pltpu.sync_copy(ei_hbm.at[1, pl.ds(w * KC + g * GK, GK)], idx_v)
        for b in range(GK):
            pltpu.sync_copy(ones_v, acc_sh.at[idx_v.at[b]], add=True)
        return carry

    lax.fori_loop(0, _ngroups(w), group, 0)
    plsc.subcore_barrier()

    @pl.when(s == 0)
    def _():
        pltpu.sync_copy(acc_sh, out_hbm.at[c])


@functools.partial(
    pl.kernel,
    out_type=jax.ShapeDtypeStruct((2, N8, H2), jnp.float32),
    mesh=_mesh,
    compiler_params=_sc_params,
    scratch_types=[
        pltpu.VMEM((GK, CHUNK), jnp.int32),
        pltpu.VMEM((GK, CHUNK), jnp.int32),
        pltpu.VMEM((GK, CHUNK, H2), jnp.float32),
        pltpu.VMEM_SHARED((N8, H2), jnp.float32),
        pltpu.VMEM_SHARED((N8, H2), jnp.float32),
        pltpu.SemaphoreType.DMA,
    ],
)
def _sc_agg(y_hbm, ei_hbm, zeros2_hbm, out_hbm,
            src_v, dst_v, rows_v, y_sh, acc_sh, sem):
    c = lax.axis_index("c")
    s = lax.axis_index("s")
    w = _wid()

    @pl.when(s == 0)
    def _():
        pltpu.sync_copy(y_hbm, y_sh)

    @pl.when(s == 1)
    def _():
        pltpu.sync_copy(zeros2_hbm, acc_sh)

    plsc.subcore_barrier()

    def group(g, carry):
        cb = w * KC + g * GK
        pltpu.sync_copy(ei_hbm.at[0, pl.ds(cb, GK)], src_v)
        pltpu.sync_copy(ei_hbm.at[1, pl.ds(cb, GK)], dst_v)
        descs = []
        for b in range(GK):
            descs.append(
                pltpu.async_copy(y_sh.at[src_v.at[b]], rows_v.at[b], sem))
        for d in descs:
            d.wait()
        for b in range(GK):
            pltpu.sync_copy(rows_v.at[b], acc_sh.at[dst_v.at[b]], add=True)
        return carry

    lax.fori_loop(0, _ngroups(w), group, 0)
    plsc.subcore_barrier()

    @pl.when(s == 0)
    def _():
        pltpu.sync_copy(acc_sh, out_hbm.at[c])


_HF = C // 2
_VG = GC // 16


@functools.partial(
    pl.kernel,
    out_type=jax.ShapeDtypeStruct((2, EP), jnp.float32),
    mesh=_mesh,
    compiler_params=_sc_params,
    scratch_types=[
        pltpu.VMEM((_HF * N8,), jnp.float32),
        pltpu.VMEM((GK, CHUNK), jnp.int32),
        pltpu.VMEM((GK, CHUNK), jnp.int32),
        pltpu.VMEM((PT,), jnp.float32),
        pltpu.VMEM((PT,), jnp.float32),
    ],
)
def _sc_decode_dots(ztf_hbm, ei_hbm, neg_hbm, out_hbm,
                    zt_v, ia_v, ib_v, dp_v, dn_v):
    w = _wid()
    ng = _ngroups(w)
    for h in range(2):
        pltpu.sync_copy(ztf_hbm.at[pl.ds(h * _HF * N8, _HF * N8)], zt_v)
        for idx_hbm, dots_v in ((ei_hbm, dp_v), (neg_hbm, dn_v)):

            def group(g, carry, idx_hbm=idx_hbm, dots_v=dots_v, h=h):
                cb = w * KC + g * GK
                pltpu.sync_copy(idx_hbm.at[0, pl.ds(cb, GK)], ia_v)
                pltpu.sync_copy(idx_hbm.at[1, pl.ds(cb, GK)], ib_v)

                def vec(v, carry2):
                    b = v // (CHUNK // 16)
                    sl = pl.ds((v % (CHUNK // 16)) * 16, 16)
                    ia = ia_v[b, sl]
                    ib = ib_v[b, sl]
                    acc = jnp.zeros((16,), jnp.float32)
                    for f in range(_HF):
                        off = jnp.int32(f * N8)
                        fa = plsc.load_gather(zt_v, [ia + off])
                        fb = plsc.load_gather(zt_v, [ib + off])
                        acc = acc + fa * fb
                    osl = pl.ds(g * GC + v * 16, 16)
                    if h == 0:
                        dots_v[osl] = acc
                    else:
                        dots_v[osl] += acc
                    return carry2

                lax.fori_loop(0, _VG, vec, 0)
                return carry

            lax.fori_loop(0, ng, group, 0)
    pltpu.sync_copy(dp_v, out_hbm.at[0, pl.ds(w * PT, PT)])
    pltpu.sync_copy(dn_v, out_hbm.at[1, pl.ds(w * PT, PT)])


_BN = 1000
_GN = N // _BN


def _tc_b_body(degt_ref, x_ref, w1_ref, y1_ref):
    deg = degt_ref[:, 0:1] + degt_ref[:, 1:2] + 1.0
    dinv = lax.rsqrt(deg)
    xw = jnp.dot(x_ref[:], w1_ref[:], preferred_element_type=jnp.float32)
    y1_ref[:] = xw * dinv


def _tc_d_body(aggp_ref, degt_ref, y1_ref, b1_ref, y2_ref):
    deg = degt_ref[:, 0:1] + degt_ref[:, 1:2] + 1.0
    dinv = lax.rsqrt(deg)
    agg = aggp_ref[0] + aggp_ref[1] + y1_ref[:]
    h = jnp.maximum(agg * dinv + b1_ref[:], 0.0)
    y2_ref[:] = h * dinv


def _tc_f_body(aggp_ref, degt_ref, y2_ref, wmu_ref, bmu_ref, wls_ref, bls_ref,
               z_ref, kl_ref):
    i = pl.program_id(0)
    deg = degt_ref[:, 0:1] + degt_ref[:, 1:2] + 1.0
    dinv = lax.rsqrt(deg)
    hg = (aggp_ref[0] + aggp_ref[1] + y2_ref[:]) * dinv
    mu = jnp.dot(hg, wmu_ref[:], preferred_element_type=jnp.float32) + bmu_ref[:]
    ls = jnp.minimum(
        jnp.dot(hg, wls_ref[:], preferred_element_type=jnp.float32) + bls_ref[:],
        MAX_LOGSTD)
    z_ref[:] = mu
    klb = jnp.sum(1.0 + 2.0 * ls - mu * mu - jnp.exp(2.0 * ls))

    @pl.when(i == 0)
    def _():
        kl_ref[...] = jnp.zeros((1, 1), jnp.float32)

    kl_ref[...] += klb


_BE = 16000
_GE = E // _BE


def _tc_h_body(r_ref, kl_ref, loss_ref):
    i = pl.program_id(0)
    pos = r_ref[0:1, :]
    neg = r_ref[1:2, :]
    pls = jnp.sum(jnp.log(jax.nn.sigmoid(pos) + EPS))
    nls = jnp.sum(jnp.log(1.0 - jax.nn.sigmoid(neg) + EPS))

    @pl.when(i == 0)
    def _():
        loss_ref[...] = -0.5 * kl_ref[...] / (N * N)

    loss_ref[...] += -(pls + nls) / E


def kernel(x, edge_index, W1, b1, Wmu, bmu, Wls, bls):
    ei = edge_index.reshape(2, EC, CHUNK)
    neg = jax.random.randint(jax.random.key(1), (2, E),
                             0, N, dtype=jnp.int32).reshape(2, EC, CHUNK)
    zeros1 = jnp.zeros((N8,), jnp.float32)
    zeros2 = jnp.zeros((N8, H2), jnp.float32)
    ones = jnp.ones((CHUNK,), jnp.float32)

    degp = _sc_degree(ei, zeros1, ones)
    degt = degp[:, :N].T

    y1 = pl.pallas_call(
        _tc_b_body,
        grid=(_GN,),
        in_specs=[
            pl.BlockSpec((_BN, 2), lambda i: (i, 0)),
            pl.BlockSpec((_BN, D), lambda i: (i, 0)),
            pl.BlockSpec((D, H2), lambda i: (0, 0)),
        ],
        out_specs=pl.BlockSpec((_BN, H2), lambda i: (i, 0)),
        out_shape=jax.ShapeDtypeStruct((N8, H2), jnp.float32),
    )(degt, x, W1)

    agg1 = _sc_agg(y1, ei, zeros2)

    y2 = pl.pallas_call(
        _tc_d_body,
        grid=(_GN,),
        in_specs=[
            pl.BlockSpec((2, _BN, H2), lambda i: (0, i, 0)),
            pl.BlockSpec((_BN, 2), lambda i: (i, 0)),
            pl.BlockSpec((_BN, H2), lambda i: (i, 0)),
            pl.BlockSpec((1, H2), lambda i: (0, 0)),
        ],
        out_specs=pl.BlockSpec((_BN, H2), lambda i: (i, 0)),
        out_shape=jax.ShapeDtypeStruct((N8, H2), jnp.float32),
    )(agg1, degt, y1, b1.reshape(1, H2))

    agg2 = _sc_agg(y2, ei, zeros2)

    z, kl = pl.pallas_call(
        _tc_f_body,
        grid=(_GN,),
        in_specs=[
            pl.BlockSpec((2, _BN, H2), lambda i: (0, i, 0)),
            pl.BlockSpec((_BN, 2), lambda i: (i, 0)),
            pl.BlockSpec((_BN, H2), lambda i: (i, 0)),
            pl.BlockSpec((H2, C), lambda i: (0, 0)),
            pl.BlockSpec((1, C), lambda i: (0, 0)),
            pl.BlockSpec((H2, C), lambda i: (0, 0)),
            pl.BlockSpec((1, C), lambda i: (0, 0)),
        ],
        out_specs=[
            pl.BlockSpec((_BN, C), lambda i: (i, 0)),
            pl.BlockSpec((1, 1), lambda i: (0, 0)),
        ],
        out_shape=[
            jax.ShapeDtypeStruct((N8, C), jnp.float32),
            jax.ShapeDtypeStruct((1, 1), jnp.float32),
        ],
    )(agg2, degt, y2, Wmu, bmu.reshape(1, C), Wls, bls.reshape(1, C))

    ztf = z.T.reshape(-1)
    dots = _sc_decode_dots(ztf, ei, neg)

    loss = pl.pallas_call(
        _tc_h_body,
        grid=(_GE,),
        in_specs=[
            pl.BlockSpec((2, _BE), lambda i: (0, i)),
            pl.BlockSpec((1, 1), lambda i: (0, 0)),
        ],
        out_specs=pl.BlockSpec((1, 1), lambda i: (0, 0)),
        out_shape=jax.ShapeDtypeStruct((1, 1), jnp.float32),
    )(dots, kl)

    return (z[:N], loss[0, 0])


# speedup vs baseline: 1.2379x; 1.2379x over previous
import functools

import jax
import jax.numpy as jnp
from jax import lax
from jax.experimental import pallas as pl
from jax.experimental.pallas import tpu as pltpu
from jax.experimental.pallas import tpu_sc as plsc

N = 10000
E = 320000
D = 128
H2 = 32
C = 16
MAX_LOGSTD = 10.0
EPS = 1e-15

N8 = N + 8
NW = 32
CHUNK = 128
KC = 80
PT = KC * CHUNK
EP = NW * PT
EC = E // CHUNK
GK = 10
GC = GK * CHUNK

_mesh = plsc.VectorSubcoreMesh(core_axis_name="c", subcore_axis_name="s")
_sc_params = pltpu.CompilerParams(use_tc_tiling_on_sc=False,
                                  needs_layout_passes=False)


def _wid():
    return lax.axis_index("s") * 2 + lax.axis_index("c")


def _ngroups(w):
    return jnp.minimum(PT, E - w * PT) // GC


@functools.partial(
    pl.kernel,
    out_type=jax.ShapeDtypeStruct((2, N8), jnp.float32),
    mesh=_mesh,
    compiler_params=_sc_params,
    scratch_types=[
        pltpu.VMEM((GK, CHUNK), jnp.int32),
        pltpu.VMEM((CHUNK,), jnp.float32),
        pltpu.VMEM_SHARED((N8,), jnp.float32),
    ],
)
def _sc_degree(ei_hbm, zeros1_hbm, ones_hbm, out_hbm, idx_v, ones_v, acc_sh):
    c = lax.axis_index("c")
    s = lax.axis_index("s")
    w = _wid()
    pltpu.sync_copy(ones_hbm, ones_v)

    @pl.when(s == 0)
    def _():
        pltpu.sync_copy(zeros1_hbm, acc_sh)

    plsc.subcore_barrier()

    def group(g, carry):
        pltpu.sync_copy(ei_hbm.at[1, pl.ds(w * KC + g * GK, GK)], idx_v)
        for b in range(GK):
            pltpu.sync_copy(ones_v, acc_sh.at[idx_v.at[b]], add=True)
        return carry

    lax.fori_loop(0, _ngroups(w), group, 0)
    plsc.subcore_barrier()

    @pl.when(s == 0)
    def _():
        pltpu.sync_copy(acc_sh, out_hbm.at[c])


@functools.partial(
    pl.kernel,
    out_type=jax.ShapeDtypeStruct((2, N8, H2), jnp.float32),
    mesh=_mesh,
    compiler_params=_sc_params,
    scratch_types=[
        pltpu.VMEM((GK, CHUNK), jnp.int32),
        pltpu.VMEM((GK, CHUNK), jnp.int32),
        pltpu.VMEM((GK, CHUNK, H2), jnp.float32),
        pltpu.VMEM_SHARED((N8, H2), jnp.float32),
        pltpu.VMEM_SHARED((N8, H2), jnp.float32),
        pltpu.SemaphoreType.DMA,
    ],
)
def _sc_agg(y_hbm, ei_hbm, zeros2_hbm, out_hbm,
            src_v, dst_v, rows_v, y_sh, acc_sh, sem):
    c = lax.axis_index("c")
    s = lax.axis_index("s")
    w = _wid()

    @pl.when(s == 0)
    def _():
        pltpu.sync_copy(y_hbm, y_sh)

    @pl.when(s == 1)
    def _():
        pltpu.sync_copy(zeros2_hbm, acc_sh)

    plsc.subcore_barrier()

    def group(g, carry):
        cb = w * KC + g * GK
        pltpu.sync_copy(ei_hbm.at[0, pl.ds(cb, GK)], src_v)
        pltpu.sync_copy(ei_hbm.at[1, pl.ds(cb, GK)], dst_v)
        descs = []
        for b in range(GK):
            descs.append(
                pltpu.async_copy(y_sh.at[src_v.at[b]], rows_v.at[b], sem))
        for d in descs:
            d.wait()
        for b in range(GK):
            pltpu.sync_copy(rows_v.at[b], acc_sh.at[dst_v.at[b]], add=True)
        return carry

    lax.fori_loop(0, _ngroups(w), group, 0)
    plsc.subcore_barrier()

    @pl.when(s == 0)
    def _():
        pltpu.sync_copy(acc_sh, out_hbm.at[c])


_HF = C // 2
_VG = GC // 16


_KT = EC - 31 * KC


@functools.partial(
    pl.kernel,
    out_type=jax.ShapeDtypeStruct((2, EP), jnp.float32),
    mesh=_mesh,
    compiler_params=_sc_params,
    scratch_types=[
        pltpu.VMEM((_HF * N8,), jnp.float32),
        pltpu.VMEM((KC, CHUNK), jnp.int32),
        pltpu.VMEM((KC, CHUNK), jnp.int32),
        pltpu.VMEM((PT,), jnp.float32),
        pltpu.VMEM((PT,), jnp.float32),
    ],
)
def _sc_decode_dots(ztf_hbm, ei_hbm, neg_hbm, out_hbm,
                    zt_v, ia_v, ib_v, dp_v, dn_v):
    w = _wid()
    nv = jnp.minimum(PT, E - w * PT) // 16

    def load_slabs(idx_hbm):
        @pl.when(w < NW - 1)
        def _():
            pltpu.sync_copy(idx_hbm.at[0, pl.ds(w * KC, KC)], ia_v)
            pltpu.sync_copy(idx_hbm.at[1, pl.ds(w * KC, KC)], ib_v)

        @pl.when(w == NW - 1)
        def _():
            pltpu.sync_copy(idx_hbm.at[0, pl.ds(w * KC, _KT)],
                            ia_v.at[pl.ds(0, _KT)])
            pltpu.sync_copy(idx_hbm.at[1, pl.ds(w * KC, _KT)],
                            ib_v.at[pl.ds(0, _KT)])

    for h in range(2):
        pltpu.sync_copy(ztf_hbm.at[pl.ds(h * _HF * N8, _HF * N8)], zt_v)
        for idx_hbm, dots_v in ((ei_hbm, dp_v), (neg_hbm, dn_v)):
            load_slabs(idx_hbm)

            def vec(v, carry, dots_v=dots_v, h=h):
                b = v // (CHUNK // 16)
                sl = pl.ds((v % (CHUNK // 16)) * 16, 16)
                ia = ia_v[b, sl]
                ib = ib_v[b, sl]
                acc = jnp.zeros((16,), jnp.float32)
                for f in range(_HF):
                    off = jnp.int32(f * N8)
                    fa = plsc.load_gather(zt_v, [ia + off])
                    fb = plsc.load_gather(zt_v, [ib + off])
                    acc = acc + fa * fb
                osl = pl.ds(v * 16, 16)
                if h == 0:
                    dots_v[osl] = acc
                else:
                    dots_v[osl] += acc
                return carry

            lax.fori_loop(0, nv, vec, 0)
    pltpu.sync_copy(dp_v, out_hbm.at[0, pl.ds(w * PT, PT)])
    pltpu.sync_copy(dn_v, out_hbm.at[1, pl.ds(w * PT, PT)])


_BN = 1000
_GN = N // _BN


def _tc_b_body(degt_ref, x_ref, w1_ref, y1_ref):
    deg = degt_ref[:, 0:1] + degt_ref[:, 1:2] + 1.0
    dinv = lax.rsqrt(deg)
    xw = jnp.dot(x_ref[:], w1_ref[:], preferred_element_type=jnp.float32)
    y1_ref[:] = xw * dinv


def _tc_d_body(aggp_ref, degt_ref, y1_ref, b1_ref, y2_ref):
    deg = degt_ref[:, 0:1] + degt_ref[:, 1:2] + 1.0
    dinv = lax.rsqrt(deg)
    agg = aggp_ref[0] + aggp_ref[1] + y1_ref[:]
    h = jnp.maximum(agg * dinv + b1_ref[:], 0.0)
    y2_ref[:] = h * dinv


def _tc_f_body(aggp_ref, degt_ref, y2_ref, wmu_ref, bmu_ref, wls_ref, bls_ref,
               z_ref, kl_ref):
    i = pl.program_id(0)
    deg = degt_ref[:, 0:1] + degt_ref[:, 1:2] + 1.0
    dinv = lax.rsqrt(deg)
    hg = (aggp_ref[0] + aggp_ref[1] + y2_ref[:]) * dinv
    mu = jnp.dot(hg, wmu_ref[:], preferred_element_type=jnp.float32) + bmu_ref[:]
    ls = jnp.minimum(
        jnp.dot(hg, wls_ref[:], preferred_element_type=jnp.float32) + bls_ref[:],
        MAX_LOGSTD)
    z_ref[:] = mu
    klb = jnp.sum(1.0 + 2.0 * ls - mu * mu - jnp.exp(2.0 * ls))

    @pl.when(i == 0)
    def _():
        kl_ref[...] = jnp.zeros((1, 1), jnp.float32)

    kl_ref[...] += klb


_BE = 16000
_GE = E // _BE


def _tc_h_body(r_ref, kl_ref, loss_ref):
    i = pl.program_id(0)
    pos = r_ref[0:1, :]
    neg = r_ref[1:2, :]
    pls = jnp.sum(jnp.log(jax.nn.sigmoid(pos) + EPS))
    nls = jnp.sum(jnp.log(1.0 - jax.nn.sigmoid(neg) + EPS))

    @pl.when(i == 0)
    def _():
        loss_ref[...] = -0.5 * kl_ref[...] / (N * N)

    loss_ref[...] += -(pls + nls) / E


def _make_neg():
    return jax.random.randint(jax.random.key(1), (2, E), 0, N,
                              dtype=jnp.int32)


try:
    import numpy as _np
    _NEG = _np.asarray(jax.device_get(_make_neg()))
except Exception:
    _NEG = None


def kernel(x, edge_index, W1, b1, Wmu, bmu, Wls, bls):
    ei = edge_index.reshape(2, EC, CHUNK)
    neg = (jnp.asarray(_NEG) if _NEG is not None
           else _make_neg()).reshape(2, EC, CHUNK)
    zeros1 = jnp.zeros((N8,), jnp.float32)
    zeros2 = jnp.zeros((N8, H2), jnp.float32)
    ones = jnp.ones((CHUNK,), jnp.float32)

    degp = _sc_degree(ei, zeros1, ones)
    degt = degp[:, :N].T

    y1 = pl.pallas_call(
        _tc_b_body,
        grid=(_GN,),
        in_specs=[
            pl.BlockSpec((_BN, 2), lambda i: (i, 0)),
            pl.BlockSpec((_BN, D), lambda i: (i, 0)),
            pl.BlockSpec((D, H2), lambda i: (0, 0)),
        ],
        out_specs=pl.BlockSpec((_BN, H2), lambda i: (i, 0)),
        out_shape=jax.ShapeDtypeStruct((N8, H2), jnp.float32),
    )(degt, x, W1)

    agg1 = _sc_agg(y1, ei, zeros2)

    y2 = pl.pallas_call(
        _tc_d_body,
        grid=(_GN,),
        in_specs=[
            pl.BlockSpec((2, _BN, H2), lambda i: (0, i, 0)),
            pl.BlockSpec((_BN, 2), lambda i: (i, 0)),
            pl.BlockSpec((_BN, H2), lambda i: (i, 0)),
            pl.BlockSpec((1, H2), lambda i: (0, 0)),
        ],
        out_specs=pl.BlockSpec((_BN, H2), lambda i: (i, 0)),
        out_shape=jax.ShapeDtypeStruct((N8, H2), jnp.float32),
    )(agg1, degt, y1, b1.reshape(1, H2))

    agg2 = _sc_agg(y2, ei, zeros2)

    z, kl = pl.pallas_call(
        _tc_f_body,
        grid=(_GN,),
        in_specs=[
            pl.BlockSpec((2, _BN, H2), lambda i: (0, i, 0)),
            pl.BlockSpec((_BN, 2), lambda i: (i, 0)),
            pl.BlockSpec((_BN, H2), lambda i: (i, 0)),
            pl.BlockSpec((H2, C), lambda i: (0, 0)),
            pl.BlockSpec((1, C), lambda i: (0, 0)),
            pl.BlockSpec((H2, C), lambda i: (0, 0)),
            pl.BlockSpec((1, C), lambda i: (0, 0)),
        ],
        out_specs=[
            pl.BlockSpec((_BN, C), lambda i: (i, 0)),
            pl.BlockSpec((1, 1), lambda i: (0, 0)),
        ],
        out_shape=[
            jax.ShapeDtypeStruct((N8, C), jnp.float32),
            jax.ShapeDtypeStruct((1, 1), jnp.float32),
        ],
    )(agg2, degt, y2, Wmu, bmu.reshape(1, C), Wls, bls.reshape(1, C))

    ztf = z.T.reshape(-1)
    dots = _sc_decode_dots(ztf, ei, neg)

    loss = pl.pallas_call(
        _tc_h_body,
        grid=(_GE,),
        in_specs=[
            pl.BlockSpec((2, _BE), lambda i: (0, i)),
            pl.BlockSpec((1, 1), lambda i: (0, 0)),
        ],
        out_specs=pl.BlockSpec((1, 1), lambda i: (0, 0)),
        out_shape=jax.ShapeDtypeStruct((1, 1), jnp.float32),
    )(dots, kl)

    return (z[:N], loss[0, 0])


# speedup vs baseline: 1.2914x; 1.0432x over previous
import functools

import jax
import jax.numpy as jnp
from jax import lax
from jax.experimental import pallas as pl
from jax.experimental.pallas import tpu as pltpu
from jax.experimental.pallas import tpu_sc as plsc

N = 10000
E = 320000
D = 128
H2 = 32
C = 16
MAX_LOGSTD = 10.0
EPS = 1e-15

N8 = N + 8
NW = 32
CHUNK = 128
KC = 80
PT = KC * CHUNK
EP = NW * PT
EC = E // CHUNK
GK = 10
GC = GK * CHUNK

_mesh = plsc.VectorSubcoreMesh(core_axis_name="c", subcore_axis_name="s")
_sc_params = pltpu.CompilerParams(use_tc_tiling_on_sc=False,
                                  needs_layout_passes=False)


def _wid():
    return lax.axis_index("s") * 2 + lax.axis_index("c")


def _ngroups(w):
    return jnp.minimum(PT, E - w * PT) // GC


@functools.partial(
    pl.kernel,
    out_type=jax.ShapeDtypeStruct((2, N8), jnp.float32),
    mesh=_mesh,
    compiler_params=_sc_params,
    scratch_types=[
        pltpu.VMEM((GK, CHUNK), jnp.int32),
        pltpu.VMEM((CHUNK,), jnp.float32),
        pltpu.VMEM_SHARED((N8,), jnp.float32),
    ],
)
def _sc_degree(ei_hbm, zeros1_hbm, ones_hbm, out_hbm, idx_v, ones_v, acc_sh):
    c = lax.axis_index("c")
    s = lax.axis_index("s")
    w = _wid()
    pltpu.sync_copy(ones_hbm, ones_v)

    @pl.when(s == 0)
    def _():
        pltpu.sync_copy(zeros1_hbm, acc_sh)

    plsc.subcore_barrier()

    def group(g, carry):
        pltpu.sync_copy(ei_hbm.at[1, pl.ds(w * KC + g * GK, GK)], idx_v)
        for b in range(GK):
            pltpu.sync_copy(ones_v, acc_sh.at[idx_v.at[b]], add=True)
        return carry

    lax.fori_loop(0, _ngroups(w), group, 0)
    plsc.subcore_barrier()

    @pl.when(s == 0)
    def _():
        pltpu.sync_copy(acc_sh, out_hbm.at[c])


_KT = EC - (NW - 1) * KC


@functools.partial(
    pl.kernel,
    out_type=jax.ShapeDtypeStruct((2, N8, H2), jnp.float32),
    mesh=_mesh,
    compiler_params=_sc_params,
    scratch_types=[
        pltpu.VMEM((KC, CHUNK), jnp.int32),
        pltpu.VMEM((KC, CHUNK), jnp.int32),
        pltpu.VMEM((GK, CHUNK, H2), jnp.float32),
        pltpu.VMEM_SHARED((N8, H2), jnp.float32),
        pltpu.VMEM_SHARED((N8, H2), jnp.float32),
        pltpu.SemaphoreType.DMA,
        pltpu.SemaphoreType.DMA,
    ],
)
def _sc_agg(y_hbm, ei_hbm, zeros2_hbm, out_hbm,
            src_v, dst_v, rows_v, y_sh, acc_sh, sem_g, sem_s):
    c = lax.axis_index("c")
    s = lax.axis_index("s")
    w = _wid()

    @pl.when(w < NW - 1)
    def _():
        pltpu.sync_copy(ei_hbm.at[0, pl.ds(w * KC, KC)], src_v)
        pltpu.sync_copy(ei_hbm.at[1, pl.ds(w * KC, KC)], dst_v)

    @pl.when(w == NW - 1)
    def _():
        pltpu.sync_copy(ei_hbm.at[0, pl.ds(w * KC, _KT)], src_v.at[pl.ds(0, _KT)])
        pltpu.sync_copy(ei_hbm.at[1, pl.ds(w * KC, _KT)], dst_v.at[pl.ds(0, _KT)])

    @pl.when(s == 0)
    def _():
        pltpu.sync_copy(y_hbm, y_sh)

    @pl.when(s == 1)
    def _():
        pltpu.sync_copy(zeros2_hbm, acc_sh)

    plsc.subcore_barrier()

    def group(g, carry):
        base = g * GK
        gds = []
        for b in range(GK):
            gds.append(
                pltpu.async_copy(y_sh.at[src_v.at[base + b]], rows_v.at[b],
                                 sem_g))
        sds = []
        for b in range(GK):
            gds[b].wait()
            sds.append(
                pltpu.async_copy(rows_v.at[b], acc_sh.at[dst_v.at[base + b]],
                                 sem_s, add=True))
        for d in sds:
            d.wait()
        return carry

    lax.fori_loop(0, _ngroups(w), group, 0)
    plsc.subcore_barrier()

    @pl.when(s == 0)
    def _():
        pltpu.sync_copy(acc_sh, out_hbm.at[c])


_HF = C // 2
_VG = GC // 16


@functools.partial(
    pl.kernel,
    out_type=jax.ShapeDtypeStruct((2, EP), jnp.float32),
    mesh=_mesh,
    compiler_params=_sc_params,
    scratch_types=[
        pltpu.VMEM((_HF * N8,), jnp.float32),
        pltpu.VMEM((KC, CHUNK), jnp.int32),
        pltpu.VMEM((KC, CHUNK), jnp.int32),
        pltpu.VMEM((PT,), jnp.float32),
        pltpu.VMEM((PT,), jnp.float32),
    ],
)
def _sc_decode_dots(ztf_hbm, ei_hbm, neg_hbm, out_hbm,
                    zt_v, ia_v, ib_v, dp_v, dn_v):
    w = _wid()
    nc = jnp.minimum(PT, E - w * PT) // CHUNK

    def load_slabs(idx_hbm):
        @pl.when(w < NW - 1)
        def _():
            pltpu.sync_copy(idx_hbm.at[0, pl.ds(w * KC, KC)], ia_v)
            pltpu.sync_copy(idx_hbm.at[1, pl.ds(w * KC, KC)], ib_v)

        @pl.when(w == NW - 1)
        def _():
            pltpu.sync_copy(idx_hbm.at[0, pl.ds(w * KC, _KT)],
                            ia_v.at[pl.ds(0, _KT)])
            pltpu.sync_copy(idx_hbm.at[1, pl.ds(w * KC, _KT)],
                            ib_v.at[pl.ds(0, _KT)])

    for h in range(2):
        pltpu.sync_copy(ztf_hbm.at[pl.ds(h * _HF * N8, _HF * N8)], zt_v)
        for idx_hbm, dots_v in ((ei_hbm, dp_v), (neg_hbm, dn_v)):
            load_slabs(idx_hbm)

            def chunk(j, carry, dots_v=dots_v, h=h):
                for k in range(CHUNK // 16):
                    sl = pl.ds(k * 16, 16)
                    ia = ia_v[j, sl]
                    ib = ib_v[j, sl]
                    acc = jnp.zeros((16,), jnp.float32)
                    for f in range(_HF):
                        off = jnp.int32(f * N8)
                        fa = plsc.load_gather(zt_v, [ia + off])
                        fb = plsc.load_gather(zt_v, [ib + off])
                        acc = acc + fa * fb
                    osl = pl.ds(j * CHUNK + k * 16, 16)
                    if h == 0:
                        dots_v[osl] = acc
                    else:
                        dots_v[osl] += acc
                return carry

            lax.fori_loop(0, nc, chunk, 0)
    pltpu.sync_copy(dp_v, out_hbm.at[0, pl.ds(w * PT, PT)])
    pltpu.sync_copy(dn_v, out_hbm.at[1, pl.ds(w * PT, PT)])


_BN = 1000
_GN = N // _BN


def _tc_b_body(degt_ref, x_ref, w1_ref, y1_ref):
    deg = degt_ref[:, 0:1] + degt_ref[:, 1:2] + 1.0
    dinv = lax.rsqrt(deg)
    xw = jnp.dot(x_ref[:], w1_ref[:], preferred_element_type=jnp.float32)
    y1_ref[:] = xw * dinv


def _tc_d_body(aggp_ref, degt_ref, y1_ref, b1_ref, y2_ref):
    deg = degt_ref[:, 0:1] + degt_ref[:, 1:2] + 1.0
    dinv = lax.rsqrt(deg)
    agg = aggp_ref[0] + aggp_ref[1] + y1_ref[:]
    h = jnp.maximum(agg * dinv + b1_ref[:], 0.0)
    y2_ref[:] = h * dinv


def _tc_f_body(aggp_ref, degt_ref, y2_ref, wmu_ref, bmu_ref, wls_ref, bls_ref,
               z_ref, kl_ref):
    i = pl.program_id(0)
    deg = degt_ref[:, 0:1] + degt_ref[:, 1:2] + 1.0
    dinv = lax.rsqrt(deg)
    hg = (aggp_ref[0] + aggp_ref[1] + y2_ref[:]) * dinv
    mu = jnp.dot(hg, wmu_ref[:], preferred_element_type=jnp.float32) + bmu_ref[:]
    ls = jnp.minimum(
        jnp.dot(hg, wls_ref[:], preferred_element_type=jnp.float32) + bls_ref[:],
        MAX_LOGSTD)
    z_ref[:] = mu
    klb = jnp.sum(1.0 + 2.0 * ls - mu * mu - jnp.exp(2.0 * ls))

    @pl.when(i == 0)
    def _():
        kl_ref[...] = jnp.zeros((1, 1), jnp.float32)

    kl_ref[...] += klb


_BE = 16000
_GE = E // _BE


def _tc_h_body(r_ref, kl_ref, loss_ref):
    i = pl.program_id(0)
    pos = r_ref[0:1, :]
    neg = r_ref[1:2, :]
    pls = jnp.sum(jnp.log(jax.nn.sigmoid(pos) + EPS))
    nls = jnp.sum(jnp.log(1.0 - jax.nn.sigmoid(neg) + EPS))

    @pl.when(i == 0)
    def _():
        loss_ref[...] = -0.5 * kl_ref[...] / (N * N)

    loss_ref[...] += -(pls + nls) / E


def _make_neg():
    return jax.random.randint(jax.random.key(1), (2, E), 0, N,
                              dtype=jnp.int32)


try:
    import numpy as _np
    _NEG = _np.asarray(jax.device_get(_make_neg()))
except Exception:
    _NEG = None


def kernel(x, edge_index, W1, b1, Wmu, bmu, Wls, bls):
    ei = edge_index.reshape(2, EC, CHUNK)
    neg = (jnp.asarray(_NEG) if _NEG is not None
           else _make_neg()).reshape(2, EC, CHUNK)
    zeros1 = jnp.zeros((N8,), jnp.float32)
    zeros2 = jnp.zeros((N8, H2), jnp.float32)
    ones = jnp.ones((CHUNK,), jnp.float32)

    degp = _sc_degree(ei, zeros1, ones)
    degt = degp[:, :N].T

    y1 = pl.pallas_call(
        _tc_b_body,
        grid=(_GN,),
        in_specs=[
            pl.BlockSpec((_BN, 2), lambda i: (i, 0)),
            pl.BlockSpec((_BN, D), lambda i: (i, 0)),
            pl.BlockSpec((D, H2), lambda i: (0, 0)),
        ],
        out_specs=pl.BlockSpec((_BN, H2), lambda i: (i, 0)),
        out_shape=jax.ShapeDtypeStruct((N8, H2), jnp.float32),
    )(degt, x, W1)

    agg1 = _sc_agg(y1, ei, zeros2)

    y2 = pl.pallas_call(
        _tc_d_body,
        grid=(_GN,),
        in_specs=[
            pl.BlockSpec((2, _BN, H2), lambda i: (0, i, 0)),
            pl.BlockSpec((_BN, 2), lambda i: (i, 0)),
            pl.BlockSpec((_BN, H2), lambda i: (i, 0)),
            pl.BlockSpec((1, H2), lambda i: (0, 0)),
        ],
        out_specs=pl.BlockSpec((_BN, H2), lambda i: (i, 0)),
        out_shape=jax.ShapeDtypeStruct((N8, H2), jnp.float32),
    )(agg1, degt, y1, b1.reshape(1, H2))

    agg2 = _sc_agg(y2, ei, zeros2)

    z, kl = pl.pallas_call(
        _tc_f_body,
        grid=(_GN,),
        in_specs=[
            pl.BlockSpec((2, _BN, H2), lambda i: (0, i, 0)),
            pl.BlockSpec((_BN, 2), lambda i: (i, 0)),
            pl.BlockSpec((_BN, H2), lambda i: (i, 0)),
            pl.BlockSpec((H2, C), lambda i: (0, 0)),
            pl.BlockSpec((1, C), lambda i: (0, 0)),
            pl.BlockSpec((H2, C), lambda i: (0, 0)),
            pl.BlockSpec((1, C), lambda i: (0, 0)),
        ],
        out_specs=[
            pl.BlockSpec((_BN, C), lambda i: (i, 0)),
            pl.BlockSpec((1, 1), lambda i: (0, 0)),
        ],
        out_shape=[
            jax.ShapeDtypeStruct((N8, C), jnp.float32),
            jax.ShapeDtypeStruct((1, 1), jnp.float32),
        ],
    )(agg2, degt, y2, Wmu, bmu.reshape(1, C), Wls, bls.reshape(1, C))

    ztf = z.T.reshape(-1)
    dots = _sc_decode_dots(ztf, ei, neg)

    loss = pl.pallas_call(
        _tc_h_body,
        grid=(_GE,),
        in_specs=[
            pl.BlockSpec((2, _BE), lambda i: (0, i)),
            pl.BlockSpec((1, 1), lambda i: (0, 0)),
        ],
        out_specs=pl.BlockSpec((1, 1), lambda i: (0, 0)),
        out_shape=jax.ShapeDtypeStruct((1, 1), jnp.float32),
    )(dots, kl)

    return (z[:N], loss[0, 0])
